# Initial kernel scaffold; baseline (speedup 1.0000x reference)
import jax
import jax.numpy as jnp
from jax.experimental import pallas as pl


def kernel(hidden_states, W1, b1, W2, b2, expert_W, expert_b):
    raise NotImplementedError("write your pallas kernel here")


# speedup vs baseline: 2.3485x; 2.3485x over previous
import functools

import jax
import jax.numpy as jnp
from jax.experimental import pallas as pl
from jax.experimental.pallas import tpu as pltpu

_B, _S, _H, _E = 4, 2048, 768, 8
_N = _B * _S
_TB = 256
_HID = _H // 2
_EPAD = 128


def _fused_body(x_ref, w1_ref, b1_ref, w2_ref, b2_ref, ew_ref, eb_ref, o_ref):
    X = x_ref[...]
    h = jnp.dot(X, w1_ref[...], preferred_element_type=jnp.float32) + b1_ref[...]
    h = jnp.maximum(h, 0.0)
    logits = jnp.dot(h, w2_ref[...], preferred_element_type=jnp.float32) + b2_ref[...]
    m = jnp.max(logits, axis=1, keepdims=True)
    p = jnp.exp(logits - m)
    probs = p / jnp.sum(p, axis=1, keepdims=True)
    idx = jnp.argmax(probs, axis=1)[:, None]
    acc = jnp.zeros((_TB, _H), jnp.float32)
    for e in range(_E):
        mask = (idx == e).astype(jnp.float32)
        y_e = jnp.dot(X, ew_ref[e], preferred_element_type=jnp.float32) + eb_ref[e][None, :]
        acc = acc + mask * y_e
    o_ref[...] = acc


def kernel(hidden_states, W1, b1, W2, b2, expert_W, expert_b):
    x2d = hidden_states.reshape(_N, _H)
    W2p = jnp.zeros((_HID, _EPAD), jnp.float32).at[:, :_E].set(W2)
    b2p = jnp.full((1, _EPAD), -1e30, jnp.float32).at[0, :_E].set(b2)
    b1r = b1.reshape(1, _HID)

    grid = (_N // _TB,)
    out = pl.pallas_call(
        _fused_body,
        grid=grid,
        in_specs=[
            pl.BlockSpec((_TB, _H), lambda i: (i, 0)),
            pl.BlockSpec((_H, _HID), lambda i: (0, 0)),
            pl.BlockSpec((1, _HID), lambda i: (0, 0)),
            pl.BlockSpec((_HID, _EPAD), lambda i: (0, 0)),
            pl.BlockSpec((1, _EPAD), lambda i: (0, 0)),
            pl.BlockSpec((_E, _H, _H), lambda i: (0, 0, 0)),
            pl.BlockSpec((_E, _H), lambda i: (0, 0)),
        ],
        out_specs=pl.BlockSpec((_TB, _H), lambda i: (i, 0)),
        out_shape=jax.ShapeDtypeStruct((_N, _H), jnp.float32),
    )(x2d, W1, b1r, W2p, b2p, expert_W, expert_b)
    return out.reshape(_B, _S, _H)
